# baseline (speedup 1.0000x reference)
import jax
import jax.numpy as jnp
from jax import lax
from jax._src.pallas import core as pl_core
from jax._src.pallas import mpmd
from jax.experimental import pallas as pl
from jax.experimental.pallas import tpu as pltpu
from jax.experimental.pallas import tpu_sc as plsc

NC = 2
NS = 16
NW = NC * NS
D = 1024
TOTAL = 32768

CHUNK = 32
NBUF = 2

S_PER = 4096
S_ROWS = NC * S_PER
T_ROWS = TOTAL - S_ROWS
PER_W = T_ROWS // NW
NCH_T = PER_W // CHUNK

IB = 512
SBLK = S_PER // IB
SCH = 64
SUBS = IB // SCH


def _tec_body(x_tec, x_scs, pe_hbm, out_hbm, idx_v, tbuf0, tbuf1,
              tg0, tg1, ts0, ts1, sidx, sidx_sp, sbuf, sg0, sg1, ss0, ss1):
    del x_scs, sidx, sidx_sp, sbuf, sg0, sg1, ss0, ss1
    bufs = (tbuf0, tbuf1)
    gsems = (tg0, tg1)
    ssems = (ts0, ts1)

    cid = lax.axis_index("c")
    sid = lax.axis_index("s")
    wid = sid * NC + cid
    base = wid * PER_W

    pltpu.sync_copy(x_tec.at[wid], idx_v)

    pltpu.async_copy(pe_hbm.at[idx_v.at[0]], bufs[0], gsems[0])

    def outer(i, carry):
        for b in range(NBUF):
            g = i * NBUF + b
            nb = (b + 1) % NBUF
            pltpu.make_async_copy(pe_hbm.at[pl.ds(0, CHUNK)], bufs[b],
                                  gsems[b]).wait()
            off = pl.multiple_of(base + g * CHUNK, CHUNK)
            pltpu.async_copy(bufs[b], out_hbm.at[pl.ds(off, CHUNK)],
                             ssems[b])
            if b < NBUF - 1:
                @pl.when(i >= 1)
                def _():
                    pltpu.make_async_copy(bufs[nb],
                                          out_hbm.at[pl.ds(0, CHUNK)],
                                          ssems[nb]).wait()

                pltpu.async_copy(pe_hbm.at[idx_v.at[g + 1]], bufs[nb],
                                 gsems[nb])
            else:
                @pl.when(g + 1 < NCH_T)
                def _():
                    pltpu.make_async_copy(bufs[nb],
                                          out_hbm.at[pl.ds(0, CHUNK)],
                                          ssems[nb]).wait()
                    pltpu.async_copy(pe_hbm.at[idx_v.at[g + 1]], bufs[nb],
                                     gsems[nb])

        return carry

    lax.fori_loop(0, NCH_T // NBUF, outer, 0)

    for b in range(NBUF):
        pltpu.make_async_copy(bufs[b], out_hbm.at[pl.ds(0, CHUNK)],
                              ssems[b]).wait()


def _scs_body(x_tec, x_scs, pe_hbm, out_hbm, idx_v, tbuf0, tbuf1,
              tg0, tg1, ts0, ts1, sidx, sidx_sp, sbuf, sg0, sg1, ss0, ss1):
    del x_tec, idx_v, tbuf0, tbuf1, tg0, tg1, ts0, ts1
    ssems = (ss0, ss1)
    sgs = (sg0, sg1)

    c = lax.axis_index("c")
    base = T_ROWS + c * S_PER

    pltpu.sync_copy(x_scs.at[c], sidx_sp)

    UN = 16

    def _issue_rows(sub, b):
        def issue(rr, carry):
            for u in range(UN):
                idx = sidx[sub * SCH + rr * UN + u]
                pltpu.async_copy(pe_hbm.at[idx], sbuf.at[b, rr * UN + u], sgs[b])
            return carry

        lax.fori_loop(0, SCH // UN, issue, 0)

    def _drain_rows(b):
        def drain(rr, carry):
            for _ in range(UN):
                pltpu.make_async_copy(pe_hbm.at[0], sbuf.at[b, 0], sgs[b]).wait()
            return carry

        lax.fori_loop(0, SCH // UN, drain, 0)

    k = 0
    for ib in range(SBLK):
        pltpu.sync_copy(sidx_sp.at[ib], sidx)
        for sub in range(SUBS):
            b = k % 2
            if k >= 2:
                pltpu.make_async_copy(sbuf.at[b],
                                      out_hbm.at[pl.ds(0, SCH)],
                                      ssems[b]).wait()
            _issue_rows(sub, b)
            if k >= 1:
                pb = 1 - b
                _drain_rows(pb)
                poff = base + (k - 1) * SCH
                pltpu.async_copy(sbuf.at[pb],
                                 out_hbm.at[pl.ds(poff, SCH)], ssems[pb])
            k += 1

    lastb = (k - 1) % 2
    _drain_rows(lastb)
    pltpu.async_copy(sbuf.at[lastb],
                     out_hbm.at[pl.ds(base + (k - 1) * SCH, SCH)],
                     ssems[lastb])
    for b in range(2):
        pltpu.make_async_copy(sbuf.at[b], out_hbm.at[pl.ds(0, SCH)],
                              ssems[b]).wait()


@jax.jit
def _sc_gather(x_tec, x_scs, pe):
    vmesh = plsc.VectorSubcoreMesh(core_axis_name="c", subcore_axis_name="s")
    smesh = plsc.ScalarSubcoreMesh(axis_name="c")
    v_vmem = pl_core.CoreMemorySpace(pltpu.MemorySpace.VMEM, vmesh)
    v_sem = pl_core.CoreMemorySpace(pltpu.MemorySpace.SEMAPHORE, vmesh)
    s_smem = pl_core.CoreMemorySpace(pltpu.MemorySpace.SMEM, smesh)
    s_sem = pl_core.CoreMemorySpace(pltpu.MemorySpace.SEMAPHORE, smesh)
    dma_dtype = pltpu.SemaphoreType.DMA.get_ref_aval().inner_aval.dtype

    scratch = [
        v_vmem((NCH_T, CHUNK), jnp.int32),
        v_vmem((CHUNK, D), jnp.float32),
        v_vmem((CHUNK, D), jnp.float32),
        v_sem((), dma_dtype),
        v_sem((), dma_dtype),
        v_sem((), dma_dtype),
        v_sem((), dma_dtype),
        s_smem((IB,), jnp.int32),
        pltpu.MemorySpace.VMEM_SHARED((SBLK, IB), jnp.int32),
        pltpu.MemorySpace.VMEM_SHARED((2, SCH, D), jnp.float32),
        s_sem((), dma_dtype),
        s_sem((), dma_dtype),
        s_sem((), dma_dtype),
        s_sem((), dma_dtype),
    ]
    run = mpmd.mpmd_map(
        [(smesh, _scs_body), (vmesh, _tec_body)],
        out_types=jax.ShapeDtypeStruct((TOTAL, D), jnp.float32),
        scratch_types=scratch,
    )
    return run(x_tec, x_scs, pe)


def kernel(x, pe):
    B, S = x.shape
    x_flat = x.reshape(B * S)
    x_tec = x_flat[:T_ROWS].reshape(NW, NCH_T, CHUNK)
    x_scs = x_flat[T_ROWS:].reshape(NC, SBLK, IB)
    out = _sc_gather(x_tec, x_scs, pe)
    return out.reshape(B, S, D)


# speedup vs baseline: 1.2537x; 1.2537x over previous
import functools

import jax
import jax.numpy as jnp
from jax import lax
from jax.experimental import pallas as pl
from jax.experimental.pallas import tpu as pltpu
from jax.experimental.pallas import tpu_sc as plsc

NC = 2
NS = 16
NW = NC * NS
D = 1024
CHUNK = 32
NBUF = 2


def _gather_body(x_hbm, pe_hbm, out_hbm, idx_v, *rest):
    nch = idx_v.shape[0]
    bufs = rest[:NBUF]
    gsems = rest[NBUF:2 * NBUF]
    ssems = rest[2 * NBUF:3 * NBUF]

    cid = lax.axis_index("c")
    sid = lax.axis_index("s")
    wid = sid * NC + cid

    pltpu.sync_copy(x_hbm.at[wid], idx_v)

    pltpu.async_copy(pe_hbm.at[idx_v.at[0]], bufs[0], gsems[0])

    def outer(i, carry):
        for b in range(NBUF):
            g = i * NBUF + b
            nb = (b + 1) % NBUF
            pltpu.make_async_copy(pe_hbm.at[pl.ds(0, CHUNK)], bufs[b],
                                  gsems[b]).wait()
            pltpu.async_copy(bufs[b], out_hbm.at[wid, g], ssems[b])
            if b < NBUF - 1:
                @pl.when(i >= 1)
                def _():
                    pltpu.make_async_copy(bufs[nb], out_hbm.at[wid, 0],
                                          ssems[nb]).wait()

                pltpu.async_copy(pe_hbm.at[idx_v.at[g + 1]], bufs[nb],
                                 gsems[nb])
            else:
                @pl.when(g + 1 < nch)
                def _():
                    pltpu.make_async_copy(bufs[nb], out_hbm.at[wid, 0],
                                          ssems[nb]).wait()
                    pltpu.async_copy(pe_hbm.at[idx_v.at[g + 1]], bufs[nb],
                                     gsems[nb])

        return carry

    lax.fori_loop(0, nch // NBUF, outer, 0)

    for b in range(NBUF):
        pltpu.make_async_copy(bufs[b], out_hbm.at[wid, 0], ssems[b]).wait()


@jax.jit
def _sc_gather(x_resh, pe):
    nch = x_resh.shape[1]
    mesh = plsc.VectorSubcoreMesh(core_axis_name="c", subcore_axis_name="s")
    scratch = (
        [pltpu.VMEM((nch, CHUNK), jnp.int32)]
        + [pltpu.VMEM((CHUNK, D), jnp.float32) for _ in range(NBUF)]
        + [pltpu.SemaphoreType.DMA for _ in range(2 * NBUF)]
    )
    run = pl.kernel(
        _gather_body,
        out_type=jax.ShapeDtypeStruct((NW, nch, CHUNK, D), jnp.float32),
        mesh=mesh,
        scratch_types=scratch,
    )
    return run(x_resh, pe)


def kernel(x, pe):
    B, S = x.shape
    total = B * S
    per_w = total // NW
    nch = per_w // CHUNK
    x_resh = x.reshape(NW, nch, CHUNK)
    out = _sc_gather(x_resh, pe)
    return out.reshape(B, S, D)
